# baseline (speedup 1.0000x reference)
import functools

import jax
import jax.numpy as jnp
from jax import lax
from jax.experimental import pallas as pl
from jax.experimental.pallas import tpu as pltpu
from jax.experimental.pallas import tpu_sc as plsc


def _make_sc_copy(m, n, l, d, dtype):
    mesh = plsc.VectorSubcoreMesh(core_axis_name="c", subcore_axis_name="s")
    nw = mesh.num_cores * mesh.num_subcores
    assert m == nw
    offset = l // 2 + l % 2
    piece = (l * d) // mesh.num_subcores

    @functools.partial(
        pl.kernel,
        out_type=jax.ShapeDtypeStruct((m * n * d,), dtype),
        mesh=mesh,
        scratch_types=[
            pltpu.VMEM_SHARED((l * d,), dtype),
            pltpu.SemaphoreType.DMA,
            pltpu.SemaphoreType.DMA,
        ],
    )
    def sc_copy(table, out, shared, ssem, wsem):
        sid = lax.axis_index("s")
        wid = sid * mesh.num_cores + lax.axis_index("c")
        pltpu.async_copy(
            table.at[pl.ds((offset - wid) * d, n * d)],
            out.at[pl.ds(wid * (n * d), n * d)],
            wsem,
        ).wait()

    return sc_copy


def kernel(q, k, embed_weight):
    m = q.shape[0]
    n = k.shape[0]
    l, d = embed_weight.shape
    flat = _make_sc_copy(m, n, l, d, embed_weight.dtype)(embed_weight.reshape(-1))
    return flat.reshape(m, n, d)


# speedup vs baseline: 34.3426x; 34.3426x over previous
import functools

import jax
import jax.numpy as jnp
from jax import lax
from jax.experimental import pallas as pl
from jax.experimental.pallas import tpu as pltpu
from jax.experimental.pallas import tpu_sc as plsc


def _make_sc_copy(m, n, l, d, dtype):
    mesh = plsc.VectorSubcoreMesh(core_axis_name="c", subcore_axis_name="s")
    nw = mesh.num_cores * mesh.num_subcores
    assert m == nw
    offset = l // 2 + l % 2
    piece = (l * d) // mesh.num_subcores
    nchunks = 8
    wchunk = (n * d) // nchunks

    @functools.partial(
        pl.kernel,
        out_type=jax.ShapeDtypeStruct((m * n * d,), dtype),
        mesh=mesh,
        scratch_types=[
            pltpu.VMEM_SHARED((l * d,), dtype),
            pltpu.SemaphoreType.DMA,
            pltpu.SemaphoreType.DMA,
        ],
    )
    def sc_copy(table, out, shared, ssem, wsem):
        sid = lax.axis_index("s")
        wid = sid * mesh.num_cores + lax.axis_index("c")
        pltpu.async_copy(
            table.at[pl.ds(sid * piece, piece)],
            shared.at[pl.ds(sid * piece, piece)],
            ssem,
        ).wait()
        plsc.subcore_barrier()
        rbase = (offset - wid) * d
        wbase = wid * (n * d)
        copies = []
        for c in range(nchunks):
            cp = pltpu.make_async_copy(
                shared.at[pl.ds(rbase + c * wchunk, wchunk)],
                out.at[pl.ds(wbase + c * wchunk, wchunk)],
                wsem,
            )
            cp.start()
            copies.append(cp)
        for cp in copies:
            cp.wait()

    return sc_copy


def kernel(q, k, embed_weight):
    m = q.shape[0]
    n = k.shape[0]
    l, d = embed_weight.shape
    flat = _make_sc_copy(m, n, l, d, embed_weight.dtype)(embed_weight.reshape(-1))
    return flat.reshape(m, n, d)


# speedup vs baseline: 35.7548x; 1.0411x over previous
import functools

import jax
import jax.numpy as jnp
from jax import lax
from jax.experimental import pallas as pl
from jax.experimental.pallas import tpu as pltpu
from jax.experimental.pallas import tpu_sc as plsc


def _make_sc_copy(m, n, l, d, dtype):
    mesh = plsc.VectorSubcoreMesh(core_axis_name="c", subcore_axis_name="s")
    nw = mesh.num_cores * mesh.num_subcores
    assert m == nw
    offset = l // 2 + l % 2
    lo = (offset - m + 1) - (offset - m + 1) % mesh.num_subcores
    nstage = l - lo
    piece = (nstage * d) // mesh.num_subcores
    nchunks = 8
    wchunk = (n * d) // nchunks

    @functools.partial(
        pl.kernel,
        out_type=jax.ShapeDtypeStruct((m * n * d,), dtype),
        mesh=mesh,
        scratch_types=[
            pltpu.VMEM_SHARED((nstage * d,), dtype),
            pltpu.SemaphoreType.DMA,
            pltpu.SemaphoreType.DMA,
        ],
    )
    def sc_copy(table, out, shared, ssem, wsem):
        sid = lax.axis_index("s")
        wid = sid * mesh.num_cores + lax.axis_index("c")
        pltpu.async_copy(
            table.at[pl.ds(lo * d + sid * piece, piece)],
            shared.at[pl.ds(sid * piece, piece)],
            ssem,
        ).wait()
        plsc.subcore_barrier()
        rbase = (offset - wid - lo) * d
        wbase = wid * (n * d)
        copies = []
        for c in range(nchunks):
            cp = pltpu.make_async_copy(
                shared.at[pl.ds(rbase + c * wchunk, wchunk)],
                out.at[pl.ds(wbase + c * wchunk, wchunk)],
                wsem,
            )
            cp.start()
            copies.append(cp)
        for cp in copies:
            cp.wait()

    return sc_copy


def kernel(q, k, embed_weight):
    m = q.shape[0]
    n = k.shape[0]
    l, d = embed_weight.shape
    flat = _make_sc_copy(m, n, l, d, embed_weight.dtype)(embed_weight.reshape(-1))
    return flat.reshape(m, n, d)


# speedup vs baseline: 42.1821x; 1.1798x over previous
import functools

import jax
import jax.numpy as jnp
from jax import lax
from jax.experimental import pallas as pl
from jax.experimental.pallas import tpu as pltpu
from jax.experimental.pallas import tpu_sc as plsc


def _make_sc_copy(m, n, l, d, dtype):
    mesh = plsc.VectorSubcoreMesh(core_axis_name="c", subcore_axis_name="s")
    nw = mesh.num_cores * mesh.num_subcores
    assert m == nw
    offset = l // 2 + l % 2
    lo = (offset - m + 1) - (offset - m + 1) % mesh.num_subcores
    nstage = l - lo
    piece = (nstage * d) // mesh.num_subcores
    nchunks = 16
    wchunk = (n * d) // nchunks
    ndma = 10

    @functools.partial(
        pl.kernel,
        out_type=jax.ShapeDtypeStruct((m * n * d,), dtype),
        mesh=mesh,
        scratch_types=[
            pltpu.VMEM_SHARED((nstage * d,), dtype),
            pltpu.VMEM((wchunk,), dtype),
            pltpu.VMEM((wchunk,), dtype),
            pltpu.SemaphoreType.DMA,
            pltpu.SemaphoreType.DMA,
            pltpu.SemaphoreType.DMA,
            pltpu.SemaphoreType.DMA,
            pltpu.SemaphoreType.DMA,
        ],
    )
    def sc_copy(table, out, shared, tb0, tb1, ssem, wsem, lsem, os0, os1):
        sid = lax.axis_index("s")
        wid = sid * mesh.num_cores + lax.axis_index("c")
        pltpu.async_copy(
            table.at[pl.ds(lo * d + sid * piece, piece)],
            shared.at[pl.ds(sid * piece, piece)],
            ssem,
        ).wait()
        plsc.subcore_barrier()
        rbase = (offset - wid - lo) * d
        wbase = wid * (n * d)
        copies = []
        for c in range(ndma):
            cp = pltpu.make_async_copy(
                shared.at[pl.ds(rbase + c * wchunk, wchunk)],
                out.at[pl.ds(wbase + c * wchunk, wchunk)],
                wsem,
            )
            cp.start()
            copies.append(cp)
        tbufs = (tb0, tb1)
        osems = (os0, os1)
        owrites = [None, None]
        for j, c in enumerate(range(ndma, nchunks)):
            b = j & 1
            if owrites[b] is not None:
                owrites[b].wait()
            pltpu.async_copy(
                shared.at[pl.ds(rbase + c * wchunk, wchunk)], tbufs[b], lsem
            ).wait()
            wr = pltpu.make_async_copy(
                tbufs[b], out.at[pl.ds(wbase + c * wchunk, wchunk)], osems[b]
            )
            wr.start()
            owrites[b] = wr
        for wr in owrites:
            if wr is not None:
                wr.wait()
        for cp in copies:
            cp.wait()

    return sc_copy


def kernel(q, k, embed_weight):
    m = q.shape[0]
    n = k.shape[0]
    l, d = embed_weight.shape
    flat = _make_sc_copy(m, n, l, d, embed_weight.dtype)(embed_weight.reshape(-1))
    return flat.reshape(m, n, d)
